# baseline (speedup 1.0000x reference)
import functools

import jax
import jax.numpy as jnp
from jax import lax
from jax.experimental import pallas as pl
from jax.experimental.pallas import tpu as pltpu
from jax.experimental.pallas import tpu_sc as plsc

N_NODES = 10000
N_EDGES = 320000
F = 128

NC = 2
NS = 16
NW = NC * NS
CHUNK = 128
CHUNKS_W = 80
E_PAD = NW * CHUNKS_W * CHUNK
ACC_ROWS = 10112
ROWS_TILE = ACC_ROWS // NS
DEG_BINS = 10240
DEG_TILE = DEG_BINS // NS

_mesh = plsc.VectorSubcoreMesh(core_axis_name="c", subcore_axis_name="s")


@functools.partial(
    pl.kernel,
    out_type=jax.ShapeDtypeStruct((NC * DEG_BINS,), jnp.float32),
    mesh=_mesh,
    scratch_types=[
        pltpu.VMEM((CHUNKS_W, CHUNK), jnp.int32),
        pltpu.VMEM((CHUNK,), jnp.float32),
        pltpu.VMEM((16,), jnp.float32),
        pltpu.VMEM((DEG_TILE,), jnp.float32),
        pltpu.VMEM_SHARED((DEG_BINS,), jnp.float32),
    ],
)
def _deg_kernel(col_hbm, deg_out, idx_v, ones_v, zero_v, rbuf_v, deg_sh):
    cid = lax.axis_index("c")
    sid = lax.axis_index("s")
    wid = cid * NS + sid
    for i in range(CHUNK // 16):
        ones_v[pl.ds(i * 16, 16)] = jnp.ones((16,), jnp.float32)
    zero_v[...] = jnp.zeros((16,), jnp.float32)

    def zbody(t, carry):
        pltpu.sync_copy(zero_v, deg_sh.at[pl.ds(sid * DEG_TILE + t * 16, 16)])
        return carry

    lax.fori_loop(0, DEG_TILE // 16, zbody, 0)
    pltpu.sync_copy(col_hbm.at[pl.ds(wid * CHUNKS_W, CHUNKS_W)], idx_v)
    plsc.subcore_barrier()

    def body(j, carry):
        pltpu.sync_copy(ones_v, deg_sh.at[idx_v.at[j]], add=True)
        return carry

    lax.fori_loop(0, CHUNKS_W, body, 0)
    plsc.subcore_barrier()
    pltpu.sync_copy(deg_sh.at[pl.ds(sid * DEG_TILE, DEG_TILE)], rbuf_v)
    pltpu.sync_copy(
        rbuf_v, deg_out.at[pl.ds(cid * DEG_BINS + sid * DEG_TILE, DEG_TILE)])


CW0 = 32
CW1 = 128
H_MAX = max(CW0, CW1) // 2


@functools.partial(
    pl.kernel,
    out_type=jax.ShapeDtypeStruct((NC, ACC_ROWS, F), jnp.float32),
    mesh=_mesh,
    scratch_types=[
        pltpu.VMEM((H_MAX, CHUNK), jnp.int32),
        pltpu.VMEM((H_MAX, CHUNK), jnp.int32),
        pltpu.VMEM((CHUNK, F), jnp.float32),
        pltpu.VMEM((CHUNK, F), jnp.float32),
        pltpu.VMEM_SHARED((ACC_ROWS, F), jnp.float32),
        pltpu.SemaphoreType.DMA,
        pltpu.SemaphoreType.DMA,
        pltpu.SemaphoreType.DMA,
        pltpu.SemaphoreType.DMA,
    ],
)
def _scatter_kernel(row_hbm, col_hbm, y_hbm, zeros_hbm, acc_out,
                    row_v, col_v, buf0, buf1, acc_sh,
                    gsem0, gsem1, ssem0, ssem1):
    cid = lax.axis_index("c")
    sid = lax.axis_index("s")
    pltpu.sync_copy(zeros_hbm, acc_sh.at[pl.ds(sid * ROWS_TILE, ROWS_TILE)])
    plsc.subcore_barrier()

    def gather(j, buf, sem):
        pltpu.async_copy(y_hbm.at[row_v.at[j]], buf, sem)

    def gwait(buf, sem):
        pltpu.make_async_copy(y_hbm.at[row_v.at[0]], buf, sem).wait()

    def scatter(j, buf, sem):
        pltpu.async_copy(buf, acc_sh.at[col_v.at[j]], sem, add=True)

    def swait(buf, sem):
        pltpu.make_async_copy(buf, acc_sh.at[col_v.at[0]], sem).wait()

    def pipeline(tile_base, n_chunks):
        n_half = n_chunks // 2
        for h in range(2):
            base = tile_base + h * n_half
            pltpu.sync_copy(row_hbm.at[pl.ds(base, n_half)],
                            row_v.at[pl.ds(0, n_half)])
            pltpu.sync_copy(col_hbm.at[pl.ds(base, n_half)],
                            col_v.at[pl.ds(0, n_half)])
            gather(0, buf0, gsem0)

            def body(t, carry):
                a = 2 * t
                b = a + 1
                gwait(buf0, gsem0)

                @pl.when(t > 0)
                def _():
                    swait(buf1, ssem1)

                gather(b, buf1, gsem1)
                scatter(a, buf0, ssem0)
                gwait(buf1, gsem1)

                @pl.when(t < n_half // 2 - 1)
                def _():
                    swait(buf0, ssem0)
                    gather(a + 2, buf0, gsem0)

                scatter(b, buf1, ssem1)
                return carry

            lax.fori_loop(0, n_half // 2, body, 0)
            swait(buf0, ssem0)
            swait(buf1, ssem1)

    @pl.when(cid == 0)
    def _():
        pipeline(sid * CW0, CW0)

    @pl.when(cid == 1)
    def _():
        pipeline(NS * CW0 + sid * CW1, CW1)

    plsc.subcore_barrier()
    pltpu.sync_copy(acc_sh.at[pl.ds(sid * ROWS_TILE, ROWS_TILE)],
                    acc_out.at[cid, pl.ds(sid * ROWS_TILE, ROWS_TILE)])


def _scale_body(x_ref, w_ref, degt_ref, y_ref, dis_ref):
    xw = jnp.dot(x_ref[...], w_ref[...], preferred_element_type=jnp.float32)
    deg = degt_ref[:N_NODES, 0:1] + degt_ref[:N_NODES, 1:2] + 1.0
    dis = lax.rsqrt(jnp.maximum(deg, 1e-12))
    dis_ref[...] = dis
    y_ref[...] = xw * dis


def _combine_body(p_ref, y_ref, dis_ref, b_ref, o_ref):
    s = p_ref[0, :N_NODES, :] + p_ref[1, :N_NODES, :] + y_ref[...]
    o_ref[...] = s * dis_ref[...] + b_ref[...]


def kernel(x, edge_index, W, b):
    ei = edge_index.astype(jnp.int32)
    n_pad = E_PAD - N_EDGES
    row_p = jnp.concatenate(
        [ei[0], jnp.zeros((n_pad,), jnp.int32)]).reshape(E_PAD // CHUNK, CHUNK)
    col_p = jnp.concatenate(
        [ei[1], jnp.full((n_pad,), N_NODES, jnp.int32)]
    ).reshape(E_PAD // CHUNK, CHUNK)

    deg1d = _deg_kernel(col_p)
    degt = jnp.stack([deg1d[:DEG_BINS], deg1d[DEG_BINS:]], axis=-1)

    y, dis = pl.pallas_call(
        _scale_body,
        out_shape=[jax.ShapeDtypeStruct((N_NODES, F), jnp.float32),
                   jax.ShapeDtypeStruct((N_NODES, 1), jnp.float32)],
    )(x, W, degt)

    zeros_a = jnp.zeros((ROWS_TILE, F), jnp.float32)
    p = _scatter_kernel(row_p, col_p, y, zeros_a)

    out = pl.pallas_call(
        _combine_body,
        out_shape=jax.ShapeDtypeStruct((N_NODES, F), jnp.float32),
    )(p, y, dis, b.reshape(1, F))
    return out


# speedup vs baseline: 1.2125x; 1.2125x over previous
import functools

import jax
import jax.numpy as jnp
from jax import lax
from jax.experimental import pallas as pl
from jax.experimental.pallas import tpu as pltpu
from jax.experimental.pallas import tpu_sc as plsc

N_NODES = 10000
N_EDGES = 320000
F = 128

NC = 2
NS = 16
NW = NC * NS
CHUNK = 128
CHUNKS_W = 80
E_PAD = NW * CHUNKS_W * CHUNK
ACC_ROWS = 10112
ROWS_TILE = ACC_ROWS // NS
DEG_BINS = 10240
DEG_TILE = DEG_BINS // NS

_mesh = plsc.VectorSubcoreMesh(core_axis_name="c", subcore_axis_name="s")


@functools.partial(
    pl.kernel,
    out_type=jax.ShapeDtypeStruct((NC * DEG_BINS,), jnp.float32),
    mesh=_mesh,
    scratch_types=[
        pltpu.VMEM((CHUNKS_W, CHUNK), jnp.int32),
        pltpu.VMEM((CHUNK,), jnp.float32),
        pltpu.VMEM((16,), jnp.float32),
        pltpu.VMEM((DEG_TILE,), jnp.float32),
        pltpu.VMEM_SHARED((DEG_BINS,), jnp.float32),
    ],
)
def _deg_kernel(col_hbm, deg_out, idx_v, ones_v, zero_v, rbuf_v, deg_sh):
    cid = lax.axis_index("c")
    sid = lax.axis_index("s")
    wid = cid * NS + sid
    for i in range(CHUNK // 16):
        ones_v[pl.ds(i * 16, 16)] = jnp.ones((16,), jnp.float32)
    zero_v[...] = jnp.zeros((16,), jnp.float32)

    def zbody(t, carry):
        pltpu.sync_copy(zero_v, deg_sh.at[pl.ds(sid * DEG_TILE + t * 16, 16)])
        return carry

    lax.fori_loop(0, DEG_TILE // 16, zbody, 0)
    pltpu.sync_copy(col_hbm.at[pl.ds(wid * CHUNKS_W, CHUNKS_W)], idx_v)
    plsc.subcore_barrier()

    def body(j, carry):
        pltpu.sync_copy(ones_v, deg_sh.at[idx_v.at[j]], add=True)
        return carry

    lax.fori_loop(0, CHUNKS_W, body, 0)
    plsc.subcore_barrier()
    pltpu.sync_copy(deg_sh.at[pl.ds(sid * DEG_TILE, DEG_TILE)], rbuf_v)
    pltpu.sync_copy(
        rbuf_v, deg_out.at[pl.ds(cid * DEG_BINS + sid * DEG_TILE, DEG_TILE)])


CW0 = 128
CW1 = 32
H_MAX = max(CW0, CW1) // 2


@functools.partial(
    pl.kernel,
    out_type=jax.ShapeDtypeStruct((NC, ACC_ROWS, F), jnp.float32),
    mesh=_mesh,
    scratch_types=[
        pltpu.VMEM((H_MAX, CHUNK), jnp.int32),
        pltpu.VMEM((H_MAX, CHUNK), jnp.int32),
        pltpu.VMEM((CHUNK, F), jnp.float32),
        pltpu.VMEM((CHUNK, F), jnp.float32),
        pltpu.VMEM_SHARED((ACC_ROWS, F), jnp.float32),
        pltpu.SemaphoreType.DMA,
        pltpu.SemaphoreType.DMA,
        pltpu.SemaphoreType.DMA,
        pltpu.SemaphoreType.DMA,
    ],
)
def _scatter_kernel(row_hbm, col_hbm, y_hbm, zeros_hbm, acc_out,
                    row_v, col_v, buf0, buf1, acc_sh,
                    gsem0, gsem1, ssem0, ssem1):
    cid = lax.axis_index("c")
    sid = lax.axis_index("s")
    pltpu.sync_copy(zeros_hbm, acc_sh.at[pl.ds(sid * ROWS_TILE, ROWS_TILE)])
    plsc.subcore_barrier()

    def gather(j, buf, sem):
        pltpu.async_copy(y_hbm.at[row_v.at[j]], buf, sem)

    def gwait(buf, sem):
        pltpu.make_async_copy(y_hbm.at[row_v.at[0]], buf, sem).wait()

    def scatter(j, buf, sem):
        pltpu.async_copy(buf, acc_sh.at[col_v.at[j]], sem, add=True)

    def swait(buf, sem):
        pltpu.make_async_copy(buf, acc_sh.at[col_v.at[0]], sem).wait()

    def pipeline(tile_base, n_chunks):
        n_half = n_chunks // 2
        for h in range(2):
            base = tile_base + h * n_half
            pltpu.sync_copy(row_hbm.at[pl.ds(base, n_half)],
                            row_v.at[pl.ds(0, n_half)])
            pltpu.sync_copy(col_hbm.at[pl.ds(base, n_half)],
                            col_v.at[pl.ds(0, n_half)])
            gather(0, buf0, gsem0)

            def body(t, carry):
                a = 2 * t
                b = a + 1
                gwait(buf0, gsem0)

                @pl.when(t > 0)
                def _():
                    swait(buf1, ssem1)

                gather(b, buf1, gsem1)
                scatter(a, buf0, ssem0)
                gwait(buf1, gsem1)

                @pl.when(t < n_half // 2 - 1)
                def _():
                    swait(buf0, ssem0)
                    gather(a + 2, buf0, gsem0)

                scatter(b, buf1, ssem1)
                return carry

            lax.fori_loop(0, n_half // 2, body, 0)
            swait(buf0, ssem0)
            swait(buf1, ssem1)

    @pl.when(cid == 0)
    def _():
        pipeline(sid * CW0, CW0)

    @pl.when(cid == 1)
    def _():
        pipeline(NS * CW0 + sid * CW1, CW1)

    plsc.subcore_barrier()
    pltpu.sync_copy(acc_sh.at[pl.ds(sid * ROWS_TILE, ROWS_TILE)],
                    acc_out.at[cid, pl.ds(sid * ROWS_TILE, ROWS_TILE)])


def _scale_body(x_ref, w_ref, degt_ref, y_ref, dis_ref):
    xw = jnp.dot(x_ref[...], w_ref[...], preferred_element_type=jnp.float32)
    deg = degt_ref[:N_NODES, 0:1] + degt_ref[:N_NODES, 1:2] + 1.0
    dis = lax.rsqrt(jnp.maximum(deg, 1e-12))
    dis_ref[...] = dis
    y_ref[...] = xw * dis


def _combine_body(p_ref, y_ref, dis_ref, b_ref, o_ref):
    s = p_ref[0, :N_NODES, :] + p_ref[1, :N_NODES, :] + y_ref[...]
    o_ref[...] = s * dis_ref[...] + b_ref[...]


def kernel(x, edge_index, W, b):
    ei = edge_index.astype(jnp.int32)
    n_pad = E_PAD - N_EDGES
    row_p = jnp.concatenate(
        [ei[0], jnp.zeros((n_pad,), jnp.int32)]).reshape(E_PAD // CHUNK, CHUNK)
    col_p = jnp.concatenate(
        [ei[1], jnp.full((n_pad,), N_NODES, jnp.int32)]
    ).reshape(E_PAD // CHUNK, CHUNK)

    deg1d = _deg_kernel(col_p)
    degt = jnp.stack([deg1d[:DEG_BINS], deg1d[DEG_BINS:]], axis=-1)

    y, dis = pl.pallas_call(
        _scale_body,
        out_shape=[jax.ShapeDtypeStruct((N_NODES, F), jnp.float32),
                   jax.ShapeDtypeStruct((N_NODES, 1), jnp.float32)],
    )(x, W, degt)

    zeros_a = jnp.zeros((ROWS_TILE, F), jnp.float32)
    p = _scatter_kernel(row_p, col_p, y, zeros_a)

    out = pl.pallas_call(
        _combine_body,
        out_shape=jax.ShapeDtypeStruct((N_NODES, F), jnp.float32),
    )(p, y, dis, b.reshape(1, F))
    return out
